# baseline (speedup 1.0000x reference)
import functools

import jax
import jax.numpy as jnp
from jax import lax
from jax.experimental import pallas as pl
from jax.experimental.pallas import tpu as pltpu
from jax.experimental.pallas import tpu_sc as plsc

D_MODEL = 2048
SEQ_LEN = 4096
BATCH = 4

NC, NS, L = 2, 16, 16
NW = NC * NS
ROWS = BATCH * SEQ_LEN
ROWS_PER_W = ROWS // NW

CHUNK = 16
CHUNK_ELEMS = CHUNK * D_MODEL
N_CHUNKS = ROWS_PER_W // CHUNK
VREGS = CHUNK_ELEMS // L
UNROLL = 16


def _body(x_hbm, pos_hbm, out_hbm, xbuf, pbuf, sem_x, sem_p):
    c = lax.axis_index("c")
    s = lax.axis_index("s")
    wid = s * NC + c
    xbase = wid * (ROWS_PER_W * D_MODEL)
    pbase = (wid * ROWS_PER_W % SEQ_LEN) * D_MODEL

    def chunk_body(g, carry):
        xo = xbase + g * CHUNK_ELEMS
        po = pbase + g * CHUNK_ELEMS
        cx = pltpu.async_copy(x_hbm.at[pl.ds(xo, CHUNK_ELEMS)], xbuf, sem_x)
        cp = pltpu.async_copy(pos_hbm.at[pl.ds(po, CHUNK_ELEMS)], pbuf, sem_p)
        cx.wait()
        cp.wait()

        def add_body(i, acc):
            base = i * (UNROLL * L)
            for j in range(UNROLL):
                o = base + j * L
                xbuf[pl.ds(o, L)] = xbuf[pl.ds(o, L)] + pbuf[pl.ds(o, L)]
            return acc

        lax.fori_loop(0, VREGS // UNROLL, add_body, 0)
        pltpu.sync_copy(xbuf, out_hbm.at[pl.ds(xo, CHUNK_ELEMS)])
        return carry

    lax.fori_loop(0, N_CHUNKS, chunk_body, 0)


_sc_add = functools.partial(
    pl.kernel,
    out_type=jax.ShapeDtypeStruct((ROWS * D_MODEL,), jnp.float32),
    mesh=plsc.VectorSubcoreMesh(core_axis_name="c", subcore_axis_name="s"),
    scratch_types=[
        pltpu.VMEM((CHUNK_ELEMS,), jnp.float32),
        pltpu.VMEM((CHUNK_ELEMS,), jnp.float32),
        pltpu.SemaphoreType.DMA,
        pltpu.SemaphoreType.DMA,
    ],
)(_body)


@jax.jit
def kernel(x, pos_table):
    xf = x.reshape(-1)
    pf = pos_table.reshape(-1)
    out = _sc_add(xf, pf)
    return out.reshape(x.shape)


# speedup vs baseline: 1.2464x; 1.2464x over previous
import functools

import jax
import jax.numpy as jnp
from jax import lax
from jax.experimental import pallas as pl
from jax.experimental.pallas import tpu as pltpu
from jax.experimental.pallas import tpu_sc as plsc

D_MODEL = 2048
SEQ_LEN = 4096
BATCH = 4

NC, NS, L = 2, 16, 16
NW = NC * NS
SEQ_PER_W = SEQ_LEN // NW

CHUNK = 8
CHUNK_ELEMS = CHUNK * D_MODEL
N_SEQ_CHUNKS = SEQ_PER_W // CHUNK
UNROLL = 16
ADD_ITERS = CHUNK_ELEMS // (UNROLL * L)


def _body(x_hbm, pos_hbm, out_hbm,
          xb0, xb1, pb0, pb1, ob0, ob1, sem_x, sem_p, sem_o):
    c = lax.axis_index("c")
    s = lax.axis_index("s")
    wid = s * NC + c
    seq0 = wid * SEQ_PER_W

    xbufs = (xb0, xb1)
    pbufs = (pb0, pb1)
    obufs = (ob0, ob1)

    def x_off(b, sc):
        return (b * SEQ_LEN + seq0 + sc * CHUNK) * D_MODEL

    def p_off(sc):
        return (seq0 + sc * CHUNK) * D_MODEL

    def start_x(b, sc, dst):
        pltpu.async_copy(x_hbm.at[pl.ds(x_off(b, sc), CHUNK_ELEMS)], dst, sem_x)

    def start_p(sc, dst):
        pltpu.async_copy(pos_hbm.at[pl.ds(p_off(sc), CHUNK_ELEMS)], dst, sem_p)

    def start_o(b, sc, src):
        pltpu.async_copy(src, out_hbm.at[pl.ds(x_off(b, sc), CHUNK_ELEMS)], sem_o)

    def wait_x(dst):
        pltpu.make_async_copy(x_hbm.at[pl.ds(0, CHUNK_ELEMS)], dst, sem_x).wait()

    def wait_p(dst):
        pltpu.make_async_copy(pos_hbm.at[pl.ds(0, CHUNK_ELEMS)], dst, sem_p).wait()

    def wait_o(src):
        pltpu.make_async_copy(src, out_hbm.at[pl.ds(0, CHUNK_ELEMS)], sem_o).wait()

    def add_chunk(xr, pr, outr):
        def body(i, acc):
            base = i * (UNROLL * L)
            for j in range(UNROLL):
                o = base + j * L
                outr[pl.ds(o, L)] = xr[pl.ds(o, L)] + pr[pl.ds(o, L)]
            return acc

        lax.fori_loop(0, ADD_ITERS, body, 0)

    start_p(0, pb0)
    start_x(0, 0, xb0)

    def sc_block(j, sc, pslot, last):
        pbuf = pbufs[pslot]
        for b in range(4):
            xbuf = xbufs[b % 2]
            obuf = obufs[b % 2]
            wait_x(xbuf)
            if b < 3:
                start_x(b + 1, sc, xbufs[(b + 1) % 2])
            elif not last:
                start_x(0, sc + 1, xbufs[0])
            else:
                @pl.when(j < (N_SEQ_CHUNKS // 2) - 1)
                def _():
                    start_x(0, sc + 1, xbufs[0])
            if b == 0:
                wait_p(pbuf)
                if not last:
                    start_p(sc + 1, pbufs[1 - pslot])
                else:
                    @pl.when(j < (N_SEQ_CHUNKS // 2) - 1)
                    def _():
                        start_p(sc + 1, pbufs[1 - pslot])
            if pslot == 0 and b < 2:
                @pl.when(j >= 1)
                def _():
                    wait_o(obuf)
            else:
                wait_o(obuf)
            add_chunk(xbuf, pbuf, obuf)
            start_o(b, sc, obuf)

    def loop_body(j, acc):
        sc_block(j, 2 * j, 0, last=False)
        sc_block(j, 2 * j + 1, 1, last=True)
        return acc

    lax.fori_loop(0, N_SEQ_CHUNKS // 2, loop_body, 0)

    wait_o(ob0)
    wait_o(ob1)


_sc_add = functools.partial(
    pl.kernel,
    out_type=jax.ShapeDtypeStruct((BATCH * SEQ_LEN * D_MODEL,), jnp.float32),
    mesh=plsc.VectorSubcoreMesh(core_axis_name="c", subcore_axis_name="s"),
    scratch_types=[
        pltpu.VMEM((CHUNK_ELEMS,), jnp.float32),
        pltpu.VMEM((CHUNK_ELEMS,), jnp.float32),
        pltpu.VMEM((CHUNK_ELEMS,), jnp.float32),
        pltpu.VMEM((CHUNK_ELEMS,), jnp.float32),
        pltpu.VMEM((CHUNK_ELEMS,), jnp.float32),
        pltpu.VMEM((CHUNK_ELEMS,), jnp.float32),
        pltpu.SemaphoreType.DMA,
        pltpu.SemaphoreType.DMA,
        pltpu.SemaphoreType.DMA,
    ],
)(_body)


@jax.jit
def kernel(x, pos_table):
    xf = x.reshape(-1)
    pf = pos_table.reshape(-1)
    out = _sc_add(xf, pf)
    return out.reshape(x.shape)


# speedup vs baseline: 3.0693x; 2.4625x over previous
import functools

import jax
import jax.numpy as jnp
from jax import lax
from jax.experimental import pallas as pl
from jax.experimental.pallas import tpu as pltpu
from jax.experimental.pallas import tpu_sc as plsc

D_MODEL = 2048
SEQ_LEN = 4096
BATCH = 4

NC, NS, L = 2, 16, 16
NW = NC * NS
SEQ_PER_W = SEQ_LEN // NW

CHUNK = 8
CHUNK_ELEMS = CHUNK * D_MODEL
N_SEQ_CHUNKS = SEQ_PER_W // CHUNK
COL_ITERS = 8
COL_UNROLL = D_MODEL // (COL_ITERS * L)


def _body(x_hbm, pos_hbm, out_hbm,
          xb0, xb1, pb0, pb1, ob0, ob1, sem_x, sem_p, sem_o):
    c = lax.axis_index("c")
    s = lax.axis_index("s")
    wid = s * NC + c
    seq0 = wid * SEQ_PER_W

    xbufs = (xb0, xb1)
    pbufs = (pb0, pb1)
    obufs = (ob0, ob1)

    def x_row(b, sc):
        return b * SEQ_LEN + seq0 + sc * CHUNK

    def p_row(sc):
        return seq0 + sc * CHUNK

    def start_x(b, sc, dst):
        pltpu.async_copy(x_hbm.at[pl.ds(x_row(b, sc), CHUNK)], dst, sem_x)

    def start_p(sc, dst):
        pltpu.async_copy(pos_hbm.at[pl.ds(p_row(sc), CHUNK)], dst, sem_p)

    def start_o(b, sc, src):
        pltpu.async_copy(src, out_hbm.at[pl.ds(x_row(b, sc), CHUNK)], sem_o)

    def wait_x(dst):
        pltpu.make_async_copy(x_hbm.at[pl.ds(0, CHUNK)], dst, sem_x).wait()

    def wait_p(dst):
        pltpu.make_async_copy(pos_hbm.at[pl.ds(0, CHUNK)], dst, sem_p).wait()

    def wait_o(src):
        pltpu.make_async_copy(src, out_hbm.at[pl.ds(0, CHUNK)], sem_o).wait()

    def add_chunk(xr, pr, outr):
        def body(i, acc):
            base = i * (COL_UNROLL * L)
            for r in range(CHUNK):
                for j in range(COL_UNROLL):
                    o = base + j * L
                    outr[r, pl.ds(o, L)] = xr[r, pl.ds(o, L)] + pr[r, pl.ds(o, L)]
            return acc

        lax.fori_loop(0, COL_ITERS, body, 0)

    start_p(0, pb0)
    start_x(0, 0, xb0)

    def sc_block(j, sc, pslot, last):
        pbuf = pbufs[pslot]
        for b in range(4):
            xbuf = xbufs[b % 2]
            obuf = obufs[b % 2]
            wait_x(xbuf)
            if b < 3:
                start_x(b + 1, sc, xbufs[(b + 1) % 2])
            elif not last:
                start_x(0, sc + 1, xbufs[0])
            else:
                @pl.when(j < (N_SEQ_CHUNKS // 2) - 1)
                def _():
                    start_x(0, sc + 1, xbufs[0])
            if b == 0:
                wait_p(pbuf)
                if not last:
                    start_p(sc + 1, pbufs[1 - pslot])
                else:
                    @pl.when(j < (N_SEQ_CHUNKS // 2) - 1)
                    def _():
                        start_p(sc + 1, pbufs[1 - pslot])
            if pslot == 0 and b < 2:
                @pl.when(j >= 1)
                def _():
                    wait_o(obuf)
            else:
                wait_o(obuf)
            add_chunk(xbuf, pbuf, obuf)
            start_o(b, sc, obuf)

    def loop_body(j, acc):
        sc_block(j, 2 * j, 0, last=False)
        sc_block(j, 2 * j + 1, 1, last=True)
        return acc

    lax.fori_loop(0, N_SEQ_CHUNKS // 2, loop_body, 0)

    wait_o(ob0)
    wait_o(ob1)


_sc_add = functools.partial(
    pl.kernel,
    out_type=jax.ShapeDtypeStruct((BATCH * SEQ_LEN, D_MODEL), jnp.float32),
    mesh=plsc.VectorSubcoreMesh(core_axis_name="c", subcore_axis_name="s"),
    scratch_types=[
        pltpu.VMEM((CHUNK, D_MODEL), jnp.float32),
        pltpu.VMEM((CHUNK, D_MODEL), jnp.float32),
        pltpu.VMEM((CHUNK, D_MODEL), jnp.float32),
        pltpu.VMEM((CHUNK, D_MODEL), jnp.float32),
        pltpu.VMEM((CHUNK, D_MODEL), jnp.float32),
        pltpu.VMEM((CHUNK, D_MODEL), jnp.float32),
        pltpu.SemaphoreType.DMA,
        pltpu.SemaphoreType.DMA,
        pltpu.SemaphoreType.DMA,
    ],
    compiler_params=pltpu.CompilerParams(use_tc_tiling_on_sc=True),
)(_body)


@jax.jit
def kernel(x, pos_table):
    x2 = x.reshape(BATCH * SEQ_LEN, D_MODEL)
    out = _sc_add(x2, pos_table)
    return out.reshape(x.shape)
